# baseline (speedup 1.0000x reference)
import functools

import jax
import jax.numpy as jnp
from jax import lax
from jax.experimental import pallas as pl
from jax.experimental.pallas import tpu as pltpu
from jax.experimental.pallas import tpu_sc as plsc

N = 10000
E = 320000
NC = 2
NT = 16
RPT = 624
NTAIL = N - NT * RPT
CHUNK = 128
EROWS = 2560
EPAD = EROWS * CHUNK - E
NTRASH = 8
NB = 4

_MESH = dict(core_axis_name="c", subcore_axis_name="s")


def _edge_pipeline(table, acc, src_hbm, dst_hbm, erow0, nch,
                   rows, srcs, dsts, gsems, ssems, isems, jsems):
    if True:
        pltpu.sync_copy(src_hbm.at[erow0], srcs[0])
        pltpu.sync_copy(dst_hbm.at[erow0], dsts[0])
        pltpu.async_copy(table.at[srcs[0]], rows[0], gsems[0])

        def _outer(o, _):
            for b in range(2):
                j = 2 * o + b
                nb = 1 - b

                @pl.when(j + 1 < nch)
                def _():
                    pltpu.sync_copy(src_hbm.at[erow0 + j + 1], srcs[nb])
                    pltpu.sync_copy(dst_hbm.at[erow0 + j + 1], dsts[nb])
                    pltpu.async_copy(table.at[srcs[nb]], rows[nb], gsems[nb])

                pltpu.make_async_copy(table.at[srcs[b]], rows[b],
                                      gsems[b]).wait()
                pltpu.sync_copy(rows[b], acc.at[dsts[b]], add=True)
            return 0

        lax.fori_loop(0, nch // 2, _outer, 0)
        return

    for k in range(4):
        pltpu.async_copy(src_hbm.at[erow0 + k], srcs[k], isems[k])
        pltpu.async_copy(dst_hbm.at[erow0 + k], dsts[k], jsems[k])
    for k in range(2):
        pltpu.make_async_copy(src_hbm.at[erow0], srcs[k], isems[k]).wait()
        pltpu.async_copy(table.at[srcs[k]], rows[k], gsems[k])

    def outer(o, _):
        for k in range(4):
            j = 4 * o + k
            b = k % 2
            pltpu.make_async_copy(table.at[srcs[k]], rows[b],
                                  gsems[b]).wait()
            pltpu.make_async_copy(dst_hbm.at[erow0], dsts[k],
                                  jsems[k]).wait()
            pltpu.async_copy(rows[b], acc.at[dsts[k]], ssems[b], add=True)

            @pl.when(j + 4 < nch)
            def _():
                pltpu.async_copy(src_hbm.at[erow0 + j + 4], srcs[k],
                                 isems[k])

            pltpu.make_async_copy(rows[b], acc.at[dsts[k]], ssems[b]).wait()

            @pl.when(j + 4 < nch)
            def _():
                pltpu.async_copy(dst_hbm.at[erow0 + j + 4], dsts[k],
                                 jsems[k])

            k2 = (k + 2) % 4

            @pl.when(j + 2 < nch)
            def _():
                pltpu.make_async_copy(src_hbm.at[erow0], srcs[k2],
                                      isems[k2]).wait()
                pltpu.async_copy(table.at[srcs[k2]], rows[b], gsems[b])

        return 0

    lax.fori_loop(0, nch // 4, outer, 0)


def _acc_writeback(acc, out_hbm, t, c, r0):
    pltpu.sync_copy(acc.at[pl.ds(r0, RPT)],
                    out_hbm.at[pl.ds(c * N + r0, RPT)])

    @pl.when(t == NT - 1)
    def _():
        pltpu.sync_copy(acc.at[pl.ds(NT * RPT, NTAIL)],
                        out_hbm.at[pl.ds(c * N + NT * RPT, NTAIL)])


_DROWS = EROWS // (NC * NT)


@functools.partial(
    pl.kernel,
    out_type=jax.ShapeDtypeStruct((NC * N, 16), jnp.float32),
    mesh=plsc.VectorSubcoreMesh(**_MESH),
    scratch_types=[
        pltpu.VMEM_SHARED((N + NTRASH, 16), jnp.float32),
        pltpu.VMEM((_DROWS, CHUNK), jnp.int32),
        pltpu.VMEM((CHUNK, 16), jnp.float32),
        pltpu.VMEM((RPT, 16), jnp.float32),
        pltpu.SemaphoreType.DMA,
    ],
)
def _deg_sc(dst_hbm, out_hbm, dacc, dst_buf, ones_v, zbuf, sem):
    c = lax.axis_index("c")
    t = lax.axis_index("s")
    one_row = jnp.where(lax.iota(jnp.int32, 16) == 0,
                        jnp.float32(1.0), jnp.float32(0.0))
    zero_row = jnp.zeros((16,), jnp.float32)

    def fill_ones(j, _):
        ones_v[j] = one_row
        return 0

    lax.fori_loop(0, CHUNK, fill_ones, 0)

    def fill_zero(j, _):
        zbuf[j] = zero_row
        return 0

    lax.fori_loop(0, RPT, fill_zero, 0)
    pltpu.sync_copy(zbuf, dacc.at[pl.ds(t * RPT, RPT)])

    @pl.when(t == NT - 1)
    def _():
        pltpu.sync_copy(zbuf.at[pl.ds(0, NTAIL)],
                        dacc.at[pl.ds(NT * RPT, NTAIL)])

    pltpu.sync_copy(dst_hbm.at[pl.ds(c * (EROWS // NC) + t * _DROWS, _DROWS)],
                    dst_buf)
    plsc.subcore_barrier()

    def fire(j, _):
        pltpu.async_copy(ones_v, dacc.at[dst_buf.at[j]], sem, add=True)
        return 0

    lax.fori_loop(0, _DROWS, fire, 0)

    def drain(j, _):
        pltpu.make_async_copy(ones_v, dacc.at[dst_buf.at[0]], sem).wait()
        return 0

    lax.fori_loop(0, _DROWS, drain, 0)
    plsc.subcore_barrier()
    _acc_writeback(dacc, out_hbm, t, c, t * RPT)


_EDGE_SCRATCH = ([pltpu.VMEM_SHARED((N + NTRASH, 128), jnp.float32)]
                 + [pltpu.VMEM((CHUNK, 128), jnp.float32)] * 2
                 + [pltpu.VMEM((CHUNK,), jnp.int32)] * 8
                 + [pltpu.SemaphoreType.DMA] * 12)


_R1 = EROWS // (NC * NT)
_R2 = EROWS // NT


@functools.partial(
    pl.kernel,
    out_type=jax.ShapeDtypeStruct((NC * N, 128), jnp.float32),
    mesh=plsc.VectorSubcoreMesh(**_MESH),
    scratch_types=_EDGE_SCRATCH,
)
def _edge_pass_l1(xws_hbm, z_hbm, src_hbm, dst_hbm, out_hbm,
                  acc, rb0, rb1, sb0, sb1, sb2, sb3, db0, db1, db2, db3,
                  g0, g1, ss0, ss1, i0, i1, i2, i3, j0, j1, j2, j3):
    c = lax.axis_index("c")
    t = lax.axis_index("s")
    r0 = t * RPT

    @pl.when(c == 0)
    def _():
        pltpu.sync_copy(xws_hbm.at[pl.ds(r0, RPT)], acc.at[pl.ds(r0, RPT)])

        @pl.when(t == NT - 1)
        def _():
            pltpu.sync_copy(xws_hbm.at[pl.ds(NT * RPT, NTAIL)],
                            acc.at[pl.ds(NT * RPT, NTAIL)])

    @pl.when(c == 1)
    def _():
        pltpu.sync_copy(z_hbm.at[pl.ds(r0, RPT)], acc.at[pl.ds(r0, RPT)])

        @pl.when(t == NT - 1)
        def _():
            pltpu.sync_copy(z_hbm.at[pl.ds(NT * RPT, NTAIL)],
                            acc.at[pl.ds(NT * RPT, NTAIL)])

    erow0 = c * (EROWS // NC) + t * _R1
    plsc.subcore_barrier()
    _edge_pipeline(xws_hbm, acc, src_hbm, dst_hbm, erow0, _R1,
                   [rb0, rb1], [sb0, sb1, sb2, sb3], [db0, db1, db2, db3],
                   [g0, g1], [ss0, ss1], [i0, i1, i2, i3], [j0, j1, j2, j3])
    plsc.subcore_barrier()
    _acc_writeback(acc, out_hbm, t, c, r0)


@functools.partial(
    pl.kernel,
    out_type=jax.ShapeDtypeStruct((NC * N, 128), jnp.float32),
    mesh=plsc.VectorSubcoreMesh(**_MESH),
    scratch_types=_EDGE_SCRATCH,
)
def _edge_pass_l2(xa_hbm, xb_hbm, src_hbm, dst_hbm, out_hbm,
                  acc, rb0, rb1, sb0, sb1, sb2, sb3, db0, db1, db2, db3,
                  g0, g1, ss0, ss1, i0, i1, i2, i3, j0, j1, j2, j3):
    c = lax.axis_index("c")
    t = lax.axis_index("s")
    r0 = t * RPT

    @pl.when(c == 0)
    def _():
        pltpu.sync_copy(xa_hbm.at[pl.ds(r0, RPT)], acc.at[pl.ds(r0, RPT)])

        @pl.when(t == NT - 1)
        def _():
            pltpu.sync_copy(xa_hbm.at[pl.ds(NT * RPT, NTAIL)],
                            acc.at[pl.ds(NT * RPT, NTAIL)])

    @pl.when(c == 1)
    def _():
        pltpu.sync_copy(xb_hbm.at[pl.ds(r0, RPT)], acc.at[pl.ds(r0, RPT)])

        @pl.when(t == NT - 1)
        def _():
            pltpu.sync_copy(xb_hbm.at[pl.ds(NT * RPT, NTAIL)],
                            acc.at[pl.ds(NT * RPT, NTAIL)])

    erow0 = t * _R2
    plsc.subcore_barrier()
    rows = [rb0, rb1]
    srcs = [sb0, sb1, sb2, sb3]
    dsts = [db0, db1, db2, db3]
    gsems = [g0, g1]
    ssems = [ss0, ss1]
    isems = [i0, i1, i2, i3]
    jsems = [j0, j1, j2, j3]

    @pl.when(c == 0)
    def _():
        _edge_pipeline(xa_hbm, acc, src_hbm, dst_hbm, erow0, _R2,
                       rows, srcs, dsts, gsems, ssems, isems, jsems)

    @pl.when(c == 1)
    def _():
        _edge_pipeline(xb_hbm, acc, src_hbm, dst_hbm, erow0, _R2,
                       rows, srcs, dsts, gsems, ssems, isems, jsems)

    plsc.subcore_barrier()
    _acc_writeback(acc, out_hbm, t, c, r0)


_JPT = 1024 // (NC * NT)


@functools.partial(
    pl.kernel,
    out_type=jax.ShapeDtypeStruct((1024,), jnp.float32),
    mesh=plsc.VectorSubcoreMesh(**_MESH),
    scratch_types=[
        pltpu.VMEM((N,), jnp.float32),
        pltpu.VMEM((_JPT,), jnp.int32),
        pltpu.VMEM((_JPT,), jnp.float32),
    ],
    compiler_params=pltpu.CompilerParams(needs_layout_passes=False),
)
def _join_sc(y_hbm, join_hbm, out_hbm, ybuf, jv, res):
    c = lax.axis_index("c")
    t = lax.axis_index("s")
    wid = t * NC + c
    pltpu.sync_copy(y_hbm, ybuf)
    pltpu.sync_copy(join_hbm.at[pl.ds(wid * _JPT, _JPT)], jv)
    for k in range(_JPT // 16):
        idx = jv[pl.ds(k * 16, 16)]
        v = plsc.load_gather(ybuf, [idx])
        res[pl.ds(k * 16, 16)] = 1.0 / (1.0 + jnp.exp(-v))
    pltpu.sync_copy(res, out_hbm.at[pl.ds(wid * _JPT, _JPT)])


_BR = 1000
_GR = N // _BR


def _dinv_of(deg_blk):
    deg = deg_blk[0, :, 0] + deg_blk[1, :, 0] + 1.0
    return lax.rsqrt(deg)


def _mm1_body(x_ref, w1_ref, deg_ref, o_ref):
    dinv = _dinv_of(deg_ref[...])
    xw = jnp.dot(x_ref[...], w1_ref[...], preferred_element_type=jnp.float32)
    o_ref[...] = xw * dinv[:, None]


def _mm2_body(o1_ref, deg_ref, b1_ref, w2_ref, oa_ref, ob_ref):
    dinv = _dinv_of(deg_ref[...])
    h = (o1_ref[0] + o1_ref[1]) * dinv[:, None]
    h = jnp.maximum(h + b1_ref[0], 0.0)
    xw = jnp.dot(h, w2_ref[...], preferred_element_type=jnp.float32)
    xws = xw * dinv[:, None]
    oa_ref[...] = xws[:, :128]
    ob_ref[...] = xws[:, 128:]


def _mm3_body(o2_ref, deg_ref, b2_ref, w3_ref, b3_ref, y_ref):
    dinv = _dinv_of(deg_ref[...])
    h = jnp.concatenate([o2_ref[0], o2_ref[1]], axis=1) * dinv[:, None]
    h = h + b2_ref[0]
    y = jnp.dot(h, w3_ref[...], preferred_element_type=jnp.float32)
    y_ref[...] = y + b3_ref[0, 0]


def _mm1(x, w1, deg2):
    return pl.pallas_call(
        _mm1_body,
        grid=(_GR,),
        in_specs=[
            pl.BlockSpec((_BR, 128), lambda i: (i, 0)),
            pl.BlockSpec((128, 128), lambda i: (0, 0)),
            pl.BlockSpec((NC, _BR, 16), lambda i: (0, i, 0)),
        ],
        out_specs=pl.BlockSpec((_BR, 128), lambda i: (i, 0)),
        out_shape=jax.ShapeDtypeStruct((N, 128), jnp.float32),
    )(x, w1, deg2)


def _mm2(out1, deg2, b1, w2):
    return pl.pallas_call(
        _mm2_body,
        grid=(_GR,),
        in_specs=[
            pl.BlockSpec((NC, _BR, 128), lambda i: (0, i, 0)),
            pl.BlockSpec((NC, _BR, 16), lambda i: (0, i, 0)),
            pl.BlockSpec((1, 128), lambda i: (0, 0)),
            pl.BlockSpec((128, 256), lambda i: (0, 0)),
        ],
        out_specs=[
            pl.BlockSpec((_BR, 128), lambda i: (i, 0)),
            pl.BlockSpec((_BR, 128), lambda i: (i, 0)),
        ],
        out_shape=[
            jax.ShapeDtypeStruct((N, 128), jnp.float32),
            jax.ShapeDtypeStruct((N, 128), jnp.float32),
        ],
    )(out1, deg2, b1, w2)


def _mm3(out2, deg2, b2, w3, b3):
    return pl.pallas_call(
        _mm3_body,
        grid=(_GR,),
        in_specs=[
            pl.BlockSpec((NC, _BR, 128), lambda i: (0, i, 0)),
            pl.BlockSpec((NC, _BR, 16), lambda i: (0, i, 0)),
            pl.BlockSpec((1, 256), lambda i: (0, 0)),
            pl.BlockSpec((256, 1), lambda i: (0, 0)),
            pl.BlockSpec((1, 1), lambda i: (0, 0)),
        ],
        out_specs=pl.BlockSpec((_BR, 1), lambda i: (i, 0)),
        out_shape=jax.ShapeDtypeStruct((N, 1), jnp.float32),
    )(out2, deg2, b2, w3, b3)


def kernel(x, edge_index, join_index, W1, b1, W2, b2, W3, b3):
    src = edge_index[0].astype(jnp.int32)
    dst = edge_index[1].astype(jnp.int32)
    join = join_index.astype(jnp.int32)

    src2d = jnp.concatenate(
        [src, jnp.zeros((EPAD,), jnp.int32)]).reshape(EROWS, CHUNK)
    dst2d = jnp.concatenate(
        [dst, jnp.full((EPAD,), N, jnp.int32)]).reshape(EROWS, CHUNK)

    deg2 = _deg_sc(dst2d).reshape(NC, N, 16)
    xws1 = _mm1(x, W1, deg2)
    zeros = jnp.zeros((N, 128), jnp.float32)
    out1 = _edge_pass_l1(xws1, zeros, src2d, dst2d).reshape(NC, N, 128)
    xws2a, xws2b = _mm2(out1, deg2, b1.reshape(1, 128), W2)
    out2 = _edge_pass_l2(xws2a, xws2b, src2d, dst2d).reshape(NC, N, 128)
    y = _mm3(out2, deg2, b2.reshape(1, 256), W3, b3.reshape(1, 1))
    z = _join_sc(y.reshape(N), join)
    return z.reshape(1024, 1)


# speedup vs baseline: 1.0302x; 1.0302x over previous
import functools

import jax
import jax.numpy as jnp
from jax import lax
from jax.experimental import pallas as pl
from jax.experimental.pallas import tpu as pltpu
from jax.experimental.pallas import tpu_sc as plsc

N = 10000
E = 320000
NC = 2
NT = 16
RPT = 624
NTAIL = N - NT * RPT
CHUNK = 128
EROWS = 2560
EPAD = EROWS * CHUNK - E
NTRASH = 8
NB = 4

_MESH = dict(core_axis_name="c", subcore_axis_name="s")


def _edge_pipeline(table, acc, src_hbm, dst_hbm, ebase, nch,
                   rows, srcs, dsts, gsems, ssems):
    pltpu.sync_copy(src_hbm.at[pl.ds(ebase, CHUNK)], srcs[0])
    pltpu.sync_copy(dst_hbm.at[pl.ds(ebase, CHUNK)], dsts[0])
    pltpu.async_copy(table.at[srcs[0]], rows[0], gsems[0])

    def _outer(o, _):
        for b in range(2):
            j = 2 * o + b
            nb = 1 - b

            @pl.when(j >= 1)
            def _():
                pltpu.make_async_copy(rows[nb], acc.at[dsts[nb]],
                                      ssems[nb]).wait()

            @pl.when(j + 1 < nch)
            def _():
                pltpu.sync_copy(
                    src_hbm.at[pl.ds(ebase + (j + 1) * CHUNK, CHUNK)],
                    srcs[nb])
                pltpu.sync_copy(
                    dst_hbm.at[pl.ds(ebase + (j + 1) * CHUNK, CHUNK)],
                    dsts[nb])
                pltpu.async_copy(table.at[srcs[nb]], rows[nb], gsems[nb])

            pltpu.make_async_copy(table.at[srcs[b]], rows[b],
                                  gsems[b]).wait()
            pltpu.async_copy(rows[b], acc.at[dsts[b]], ssems[b], add=True)
        return 0

    lax.fori_loop(0, nch // 2, _outer, 0)
    pltpu.make_async_copy(rows[1], acc.at[dsts[1]], ssems[1]).wait()


def _acc_writeback(acc, out_hbm, t, c, r0):
    pltpu.sync_copy(acc.at[pl.ds(r0, RPT)],
                    out_hbm.at[pl.ds(c * N + r0, RPT)])

    @pl.when(t == NT - 1)
    def _():
        pltpu.sync_copy(acc.at[pl.ds(NT * RPT, NTAIL)],
                        out_hbm.at[pl.ds(c * N + NT * RPT, NTAIL)])


_DROWS = EROWS // (NC * NT)


@functools.partial(
    pl.kernel,
    out_type=jax.ShapeDtypeStruct((NC * N, 16), jnp.float32),
    mesh=plsc.VectorSubcoreMesh(**_MESH),
    scratch_types=[
        pltpu.VMEM_SHARED((N + NTRASH, 16), jnp.float32),
        pltpu.VMEM((_DROWS, CHUNK), jnp.int32),
        pltpu.VMEM((CHUNK, 16), jnp.float32),
        pltpu.VMEM((RPT, 16), jnp.float32),
        pltpu.SemaphoreType.DMA,
    ],
)
def _deg_sc(dst_hbm, out_hbm, dacc, dst_buf, ones_v, zbuf, sem):
    c = lax.axis_index("c")
    t = lax.axis_index("s")
    one_row = jnp.where(lax.iota(jnp.int32, 16) == 0,
                        jnp.float32(1.0), jnp.float32(0.0))
    zero_row = jnp.zeros((16,), jnp.float32)

    def fill_ones(j, _):
        ones_v[j] = one_row
        return 0

    lax.fori_loop(0, CHUNK, fill_ones, 0)

    def fill_zero(j, _):
        zbuf[j] = zero_row
        return 0

    lax.fori_loop(0, RPT, fill_zero, 0)
    pltpu.sync_copy(zbuf, dacc.at[pl.ds(t * RPT, RPT)])

    @pl.when(t == NT - 1)
    def _():
        pltpu.sync_copy(zbuf.at[pl.ds(0, NTAIL)],
                        dacc.at[pl.ds(NT * RPT, NTAIL)])

    pltpu.sync_copy(dst_hbm.at[pl.ds(c * (EROWS // NC) + t * _DROWS, _DROWS)],
                    dst_buf)
    plsc.subcore_barrier()

    def fire(j, _):
        pltpu.async_copy(ones_v, dacc.at[dst_buf.at[j]], sem, add=True)
        return 0

    lax.fori_loop(0, _DROWS, fire, 0)

    def drain(j, _):
        pltpu.make_async_copy(ones_v, dacc.at[dst_buf.at[0]], sem).wait()
        return 0

    lax.fori_loop(0, _DROWS, drain, 0)
    plsc.subcore_barrier()
    _acc_writeback(dacc, out_hbm, t, c, t * RPT)


_EDGE_SCRATCH = ([pltpu.VMEM_SHARED((N + NTRASH, 128), jnp.float32)]
                 + [pltpu.VMEM((CHUNK, 128), jnp.float32)] * 2
                 + [pltpu.VMEM((CHUNK,), jnp.int32)] * 4
                 + [pltpu.SemaphoreType.DMA] * 4)


_R1 = EROWS // (NC * NT)
_R2 = EROWS // NT


@functools.partial(
    pl.kernel,
    out_type=jax.ShapeDtypeStruct((NC * N, 128), jnp.float32),
    mesh=plsc.VectorSubcoreMesh(**_MESH),
    scratch_types=_EDGE_SCRATCH,
)
def _edge_pass_l1(xws_hbm, z_hbm, src_hbm, dst_hbm, out_hbm,
                  acc, rb0, rb1, sb0, sb1, db0, db1, g0, g1, ss0, ss1):
    c = lax.axis_index("c")
    t = lax.axis_index("s")
    r0 = t * RPT

    @pl.when(c == 0)
    def _():
        pltpu.sync_copy(xws_hbm.at[pl.ds(r0, RPT)], acc.at[pl.ds(r0, RPT)])

        @pl.when(t == NT - 1)
        def _():
            pltpu.sync_copy(xws_hbm.at[pl.ds(NT * RPT, NTAIL)],
                            acc.at[pl.ds(NT * RPT, NTAIL)])

    @pl.when(c == 1)
    def _():
        pltpu.sync_copy(z_hbm.at[pl.ds(r0, RPT)], acc.at[pl.ds(r0, RPT)])

        @pl.when(t == NT - 1)
        def _():
            pltpu.sync_copy(z_hbm.at[pl.ds(NT * RPT, NTAIL)],
                            acc.at[pl.ds(NT * RPT, NTAIL)])

    ebase = (c * (EROWS // NC) + t * _R1) * CHUNK
    plsc.subcore_barrier()
    _edge_pipeline(xws_hbm, acc, src_hbm, dst_hbm, ebase, _R1,
                   [rb0, rb1], [sb0, sb1], [db0, db1], [g0, g1], [ss0, ss1])
    plsc.subcore_barrier()
    _acc_writeback(acc, out_hbm, t, c, r0)


@functools.partial(
    pl.kernel,
    out_type=jax.ShapeDtypeStruct((NC * N, 128), jnp.float32),
    mesh=plsc.VectorSubcoreMesh(**_MESH),
    scratch_types=_EDGE_SCRATCH,
)
def _edge_pass_l2(xa_hbm, xb_hbm, src_hbm, dst_hbm, out_hbm,
                  acc, rb0, rb1, sb0, sb1, db0, db1, g0, g1, ss0, ss1):
    c = lax.axis_index("c")
    t = lax.axis_index("s")
    r0 = t * RPT

    @pl.when(c == 0)
    def _():
        pltpu.sync_copy(xa_hbm.at[pl.ds(r0, RPT)], acc.at[pl.ds(r0, RPT)])

        @pl.when(t == NT - 1)
        def _():
            pltpu.sync_copy(xa_hbm.at[pl.ds(NT * RPT, NTAIL)],
                            acc.at[pl.ds(NT * RPT, NTAIL)])

    @pl.when(c == 1)
    def _():
        pltpu.sync_copy(xb_hbm.at[pl.ds(r0, RPT)], acc.at[pl.ds(r0, RPT)])

        @pl.when(t == NT - 1)
        def _():
            pltpu.sync_copy(xb_hbm.at[pl.ds(NT * RPT, NTAIL)],
                            acc.at[pl.ds(NT * RPT, NTAIL)])

    ebase = t * _R2 * CHUNK
    plsc.subcore_barrier()
    rows = [rb0, rb1]
    srcs = [sb0, sb1]
    dsts = [db0, db1]
    gsems = [g0, g1]
    ssems = [ss0, ss1]

    @pl.when(c == 0)
    def _():
        _edge_pipeline(xa_hbm, acc, src_hbm, dst_hbm, ebase, _R2,
                       rows, srcs, dsts, gsems, ssems)

    @pl.when(c == 1)
    def _():
        _edge_pipeline(xb_hbm, acc, src_hbm, dst_hbm, ebase, _R2,
                       rows, srcs, dsts, gsems, ssems)

    plsc.subcore_barrier()
    _acc_writeback(acc, out_hbm, t, c, r0)


_JPT = 1024 // (NC * NT)


@functools.partial(
    pl.kernel,
    out_type=jax.ShapeDtypeStruct((1024,), jnp.float32),
    mesh=plsc.VectorSubcoreMesh(**_MESH),
    scratch_types=[
        pltpu.VMEM((N,), jnp.float32),
        pltpu.VMEM((_JPT,), jnp.int32),
        pltpu.VMEM((_JPT,), jnp.float32),
    ],
    compiler_params=pltpu.CompilerParams(needs_layout_passes=False),
)
def _join_sc(y_hbm, join_hbm, out_hbm, ybuf, jv, res):
    c = lax.axis_index("c")
    t = lax.axis_index("s")
    wid = t * NC + c
    pltpu.sync_copy(y_hbm, ybuf)
    pltpu.sync_copy(join_hbm.at[pl.ds(wid * _JPT, _JPT)], jv)
    for k in range(_JPT // 16):
        idx = jv[pl.ds(k * 16, 16)]
        v = plsc.load_gather(ybuf, [idx])
        res[pl.ds(k * 16, 16)] = 1.0 / (1.0 + jnp.exp(-v))
    pltpu.sync_copy(res, out_hbm.at[pl.ds(wid * _JPT, _JPT)])


_BR = 1000
_GR = N // _BR


def _dinv_of(deg_blk):
    deg = deg_blk[0, :, 0] + deg_blk[1, :, 0] + 1.0
    return lax.rsqrt(deg)


def _mm1_body(x_ref, w1_ref, deg_ref, o_ref):
    dinv = _dinv_of(deg_ref[...])
    xw = jnp.dot(x_ref[...], w1_ref[...], preferred_element_type=jnp.float32)
    o_ref[...] = xw * dinv[:, None]


def _mm2_body(o1_ref, deg_ref, b1_ref, w2_ref, oa_ref, ob_ref):
    dinv = _dinv_of(deg_ref[...])
    h = (o1_ref[0] + o1_ref[1]) * dinv[:, None]
    h = jnp.maximum(h + b1_ref[0], 0.0)
    xw = jnp.dot(h, w2_ref[...], preferred_element_type=jnp.float32)
    xws = xw * dinv[:, None]
    oa_ref[...] = xws[:, :128]
    ob_ref[...] = xws[:, 128:]


def _mm3_body(o2_ref, deg_ref, b2_ref, w3_ref, b3_ref, y_ref):
    dinv = _dinv_of(deg_ref[...])
    h = jnp.concatenate([o2_ref[0], o2_ref[1]], axis=1) * dinv[:, None]
    h = h + b2_ref[0]
    y = jnp.dot(h, w3_ref[...], preferred_element_type=jnp.float32)
    y_ref[...] = y + b3_ref[0, 0]


def _mm1(x, w1, deg2):
    return pl.pallas_call(
        _mm1_body,
        grid=(_GR,),
        in_specs=[
            pl.BlockSpec((_BR, 128), lambda i: (i, 0)),
            pl.BlockSpec((128, 128), lambda i: (0, 0)),
            pl.BlockSpec((NC, _BR, 16), lambda i: (0, i, 0)),
        ],
        out_specs=pl.BlockSpec((_BR, 128), lambda i: (i, 0)),
        out_shape=jax.ShapeDtypeStruct((N, 128), jnp.float32),
    )(x, w1, deg2)


def _mm2(out1, deg2, b1, w2):
    return pl.pallas_call(
        _mm2_body,
        grid=(_GR,),
        in_specs=[
            pl.BlockSpec((NC, _BR, 128), lambda i: (0, i, 0)),
            pl.BlockSpec((NC, _BR, 16), lambda i: (0, i, 0)),
            pl.BlockSpec((1, 128), lambda i: (0, 0)),
            pl.BlockSpec((128, 256), lambda i: (0, 0)),
        ],
        out_specs=[
            pl.BlockSpec((_BR, 128), lambda i: (i, 0)),
            pl.BlockSpec((_BR, 128), lambda i: (i, 0)),
        ],
        out_shape=[
            jax.ShapeDtypeStruct((N, 128), jnp.float32),
            jax.ShapeDtypeStruct((N, 128), jnp.float32),
        ],
    )(out1, deg2, b1, w2)


def _mm3(out2, deg2, b2, w3, b3):
    return pl.pallas_call(
        _mm3_body,
        grid=(_GR,),
        in_specs=[
            pl.BlockSpec((NC, _BR, 128), lambda i: (0, i, 0)),
            pl.BlockSpec((NC, _BR, 16), lambda i: (0, i, 0)),
            pl.BlockSpec((1, 256), lambda i: (0, 0)),
            pl.BlockSpec((256, 1), lambda i: (0, 0)),
            pl.BlockSpec((1, 1), lambda i: (0, 0)),
        ],
        out_specs=pl.BlockSpec((_BR, 1), lambda i: (i, 0)),
        out_shape=jax.ShapeDtypeStruct((N, 1), jnp.float32),
    )(out2, deg2, b2, w3, b3)


def kernel(x, edge_index, join_index, W1, b1, W2, b2, W3, b3):
    src = edge_index[0].astype(jnp.int32)
    dst = edge_index[1].astype(jnp.int32)
    join = join_index.astype(jnp.int32)

    src1 = jnp.concatenate([src, jnp.zeros((EPAD,), jnp.int32)])
    dst1 = jnp.concatenate([dst, jnp.full((EPAD,), N, jnp.int32)])
    dst2d = dst1.reshape(EROWS, CHUNK)

    deg2 = _deg_sc(dst2d).reshape(NC, N, 16)
    xws1 = _mm1(x, W1, deg2)
    zeros = jnp.zeros((N, 128), jnp.float32)
    out1 = _edge_pass_l1(xws1, zeros, src1, dst1).reshape(NC, N, 128)
    xws2a, xws2b = _mm2(out1, deg2, b1.reshape(1, 128), W2)
    out2 = _edge_pass_l2(xws2a, xws2b, src1, dst1).reshape(NC, N, 128)
    y = _mm3(out2, deg2, b2.reshape(1, 256), W3, b3.reshape(1, 1))
    z = _join_sc(y.reshape(N), join)
    return z.reshape(1024, 1)
